# baseline (speedup 1.0000x reference)
import functools

import jax
import jax.numpy as jnp
from jax import lax
from jax.experimental import pallas as pl
from jax.experimental.pallas import tpu as pltpu
from jax.experimental.pallas import tpu_sc as plsc

_LANES = 16
_CHUNK = 128


@functools.lru_cache(maxsize=None)
def _make_sc_gather(B, H, U, I):
    info = plsc.get_sparse_core_info()
    NC, NS = info.num_cores, info.num_subcores
    NW = NC * NS
    bpw = B // NW
    nch = bpw // _CHUNK
    rows_2d = B // _CHUNK
    assert bpw * NW == B and nch * _CHUNK == bpw

    mesh = plsc.VectorSubcoreMesh(core_axis_name="c", subcore_axis_name="s")

    @functools.partial(
        pl.kernel,
        mesh=mesh,
        compiler_params=pltpu.CompilerParams(use_tc_tiling_on_sc=False),
        out_type=(
            jax.ShapeDtypeStruct((B, H), jnp.float32),
            jax.ShapeDtypeStruct((B, H), jnp.float32),
        ),
        scratch_types=[
            pltpu.VMEM((nch, _CHUNK), jnp.int32),
            pltpu.VMEM((nch, _CHUNK), jnp.int32),
            pltpu.VMEM((bpw, H), jnp.float32),
            pltpu.VMEM((bpw, H), jnp.float32),
            pltpu.SemaphoreType.DMA,
            pltpu.SemaphoreType.DMA,
        ],
    )
    def sc_gather(uid_hbm, iid_hbm, uemb_hbm, iemb_hbm, ue_out, ie_out,
                  uidx, iidx, urows, irows, su, si):
        wid = lax.axis_index("s") * NC + lax.axis_index("c")
        row0 = wid * nch
        base = wid * bpw

        pltpu.sync_copy(uid_hbm.at[pl.ds(row0, nch)], uidx)
        pltpu.sync_copy(iid_hbm.at[pl.ds(row0, nch)], iidx)

        for r in range(nch):
            for j in range(_CHUNK // _LANES):
                sl = pl.ds(j * _LANES, _LANES)
                v = uidx[r, sl] - 1
                uidx[r, sl] = jnp.where(v < 0, v + U, v)
                w = iidx[r, sl] - 1
                iidx[r, sl] = jnp.where(w < 0, w + I, w)

        copies = []
        for r in range(nch):
            dst = pl.ds(r * _CHUNK, _CHUNK)
            copies.append(pltpu.async_copy(uemb_hbm.at[uidx.at[r]],
                                           urows.at[dst], su))
            copies.append(pltpu.async_copy(iemb_hbm.at[iidx.at[r]],
                                           irows.at[dst], si))
        for c in copies:
            c.wait()

        pltpu.sync_copy(urows, ue_out.at[pl.ds(base, bpw)])
        pltpu.sync_copy(irows, ie_out.at[pl.ds(base, bpw)])

    return sc_gather


def _mlp_body(ue_ref, ie_ref, tp_ref, g_ref, o_ref,
              gemb_ref, oemb_ref,
              w1u_ref, w1g_ref, w1o_ref, w1i_ref, w1t_ref, b1_ref,
              w2_ref, b2_ref, w3_ref, b3_ref, out_ref):
    f32 = jnp.float32
    dot = functools.partial(jnp.dot, preferred_element_type=f32)

    acc = dot(ue_ref[...], w1u_ref[...])
    acc += dot(ie_ref[...], w1i_ref[...])
    acc += dot(tp_ref[...], w1t_ref[...])

    g2 = dot(gemb_ref[...], w1g_ref[...])
    acc += jnp.where(g_ref[...] == 0, g2[0:1, :], g2[1:2, :])

    o2 = dot(oemb_ref[...], w1o_ref[...])
    blk = o_ref.shape[0]
    iota = lax.broadcasted_iota(jnp.int32, (blk, o2.shape[0]), 1)
    oh = (o_ref[...] == iota).astype(f32)
    acc += dot(oh, o2)

    h1 = jnp.maximum(acc + b1_ref[...], 0.0)
    h2 = jnp.maximum(dot(h1, w2_ref[...]) + b2_ref[...], 0.0)
    z = dot(h2, w3_ref[...]) + b3_ref[...]
    out_ref[...] = 1.0 / (1.0 + jnp.exp(-z))


def _mlp_call(B, blk, ue, ie, tp, g2d, o2d, gemb, oemb,
              w1u, w1g, w1o, w1i, w1t, b1, w2, b2, w3, b3):
    grid = (B // blk,)

    def row_spec(c):
        return pl.BlockSpec((blk, c), lambda i: (i, 0))

    def full_spec(shape):
        return pl.BlockSpec(shape, lambda i: (0,) * len(shape))

    return pl.pallas_call(
        _mlp_body,
        grid=grid,
        in_specs=[
            row_spec(ue.shape[1]), row_spec(ie.shape[1]), row_spec(tp.shape[1]),
            row_spec(1), row_spec(1),
            full_spec(gemb.shape), full_spec(oemb.shape),
            full_spec(w1u.shape), full_spec(w1g.shape), full_spec(w1o.shape),
            full_spec(w1i.shape), full_spec(w1t.shape), full_spec(b1.shape),
            full_spec(w2.shape), full_spec(b2.shape),
            full_spec(w3.shape), full_spec(b3.shape),
        ],
        out_specs=pl.BlockSpec((blk, 1), lambda i: (i, 0)),
        out_shape=jax.ShapeDtypeStruct((B, 1), jnp.float32),
    )(ue, ie, tp, g2d, o2d, gemb, oemb,
      w1u, w1g, w1o, w1i, w1t, b1, w2, b2, w3, b3)


def kernel(x, gender, occupation, type, user_emb, item_emb, gender_emb, occ_emb,
           W1, b1, W2, b2, W3, b3):
    B = x.shape[0]
    U, H = user_emb.shape
    I = item_emb.shape[0]

    uid2d = x[:, 0].astype(jnp.int32).reshape(B // _CHUNK, _CHUNK)
    iid2d = x[:, 1].astype(jnp.int32).reshape(B // _CHUNK, _CHUNK)
    ue, ie = _make_sc_gather(B, H, U, I)(uid2d, iid2d, user_emb, item_emb)

    Hg = gender_emb.shape[1]
    Ho = occ_emb.shape[1]
    o0 = H + Hg
    i0 = o0 + Ho
    t0 = i0 + H
    w1u = W1[:H]
    w1g = W1[H:o0]
    w1o = W1[o0:i0]
    w1i = W1[i0:t0]
    w1t = W1[t0:]

    return _mlp_call(
        B, 2048, ue, ie, type,
        gender.astype(jnp.int32).reshape(B, 1),
        occupation.astype(jnp.int32).reshape(B, 1),
        gender_emb, occ_emb,
        w1u, w1g, w1o, w1i, w1t, b1.reshape(1, H),
        W2, b2.reshape(1, -1), W3, b3.reshape(1, 1))
